# baseline (speedup 1.0000x reference)
import functools

import jax
import jax.numpy as jnp
from jax import lax
from jax.experimental import pallas as pl
from jax.experimental.pallas import tpu as pltpu
from jax.experimental.pallas import tpu_sc as plsc

_NC = 2
_NS = 16
_NW = _NC * _NS

_NPARTS_SC = 2
_NPARTS_TC = 8
_GATHER_CHUNK = 32
_NBUF = 4
_DEPTH = 2


def _sc_gather_part(table, tids_part):
    b_sz, s_chunk = tids_part.shape
    d = table.shape[1]
    n_p = b_sz * s_chunk
    rpw = n_p // _NW
    wpb = _NW // b_sz
    ch = min(_GATHER_CHUNK, rpw)
    nch = rpw // ch
    assert rpw * _NW == n_p and ch * nch == rpw

    mesh = plsc.VectorSubcoreMesh(
        core_axis_name="c", subcore_axis_name="s",
        num_cores=_NC, num_subcores=_NS,
    )

    @functools.partial(
        pl.kernel,
        mesh=mesh,
        out_type=jax.ShapeDtypeStruct((n_p, d), jnp.float32),
        scratch_types=(
            [pltpu.VMEM((rpw,), jnp.int32)]
            + [pltpu.VMEM((ch, d), jnp.float32) for _ in range(_NBUF)]
            + [pltpu.SemaphoreType.DMA for _ in range(2 * _NBUF)]
        ),
    )
    def k(table_hbm, tids_hbm, out_hbm, idx_v, *bufs):
        rows = bufs[:_NBUF]
        gsem = bufs[_NBUF:2 * _NBUF]
        ssem = bufs[2 * _NBUF:]
        wid = lax.axis_index("s") * _NC + lax.axis_index("c")
        bi = wid // wpb
        si = (wid % wpb) * rpw
        base = wid * rpw
        pltpu.sync_copy(tids_hbm.at[bi, pl.ds(si, rpw)], idx_v)

        def gather(ci):
            b = ci % _NBUF
            return pltpu.async_copy(
                table_hbm.at[idx_v.at[pl.ds(ci * ch, ch)]], rows[b], gsem[b])

        def store(ci):
            b = ci % _NBUF
            return pltpu.async_copy(
                rows[b], out_hbm.at[pl.ds(base + ci * ch, ch)], ssem[b])

        depth = min(_DEPTH, nch)
        gcp = [None] * _NBUF
        scp = [None] * _NBUF
        for ci in range(depth):
            gcp[ci % _NBUF] = gather(ci)
        for ci in range(nch):
            b = ci % _NBUF
            gcp[b].wait()
            scp[b] = store(ci)
            nxt = ci + depth
            if nxt < nch:
                nb = nxt % _NBUF
                if scp[nb] is not None:
                    scp[nb].wait()
                gcp[nb] = gather(nxt)
        for ci in range(max(0, nch - _NBUF), nch):
            scp[ci % _NBUF].wait()

    return k(table, tids_part)


def _tc_add_layernorm_part(tok, pos_table, seg_table, sid3, gamma2d, beta2d,
                           eps, prev, h, s_chunk, total_n, b_sz):
    n_p, d = tok.shape
    s_len = pos_table.shape[0]
    br = s_chunk
    assert n_p % br == 0
    sc_chunk = n_p // b_sz
    tpb = sc_chunk // br
    spb = s_len // br
    nblk = b_sz

    def out_map(i):
        return (i * spb + h, 0)

    def body(tok_ref, pos_ref, seg_ref, sid_ref, g_ref, b_ref, *rest):
        o_ref = rest[-1]
        s0 = seg_ref[0, :]
        sd = seg_ref[1, :] - s0
        sid_col = sid_ref[0, 0, :].astype(jnp.float32).reshape(br, 1)
        x = tok_ref[...] + pos_ref[...] + s0[None, :] + sid_col * sd[None, :]
        mean = jnp.mean(x, axis=-1, keepdims=True)
        xc = x - mean
        var = jnp.mean(xc * xc, axis=-1, keepdims=True)
        inv = lax.rsqrt(var + eps)
        o_ref[...] = xc * inv * g_ref[...] + b_ref[...]

    in_specs = [
        pl.BlockSpec((br, d), lambda i: (i * tpb + h % tpb, 0)),
        pl.BlockSpec((br, d), lambda i: (h, 0)),
        pl.BlockSpec((8, d), lambda i: (0, 0)),
        pl.BlockSpec((1, 1, br), lambda i: (out_map(i)[0], 0, 0)),
        pl.BlockSpec((1, d), lambda i: (0, 0)),
        pl.BlockSpec((1, d), lambda i: (0, 0)),
    ]
    args = [tok, pos_table, seg_table, sid3, gamma2d, beta2d]
    aliases = {}
    if prev is not None:
        in_specs.append(pl.BlockSpec(memory_space=pl.ANY))
        args.append(prev)
        aliases = {6: 0}

    return pl.pallas_call(
        body,
        grid=(nblk,),
        in_specs=in_specs,
        out_specs=pl.BlockSpec((br, d), out_map),
        out_shape=jax.ShapeDtypeStruct((total_n, d), jnp.float32),
        input_output_aliases=aliases,
    )(*args)


def kernel(token_ids, segment_ids, input_ids, token_table, segment_table,
           position_table, ln_gamma, ln_beta):
    b, s = input_ids.shape
    d = token_table.shape[1]
    n = b * s
    sc_chunk = s // _NPARTS_SC
    s_chunk = s // _NPARTS_TC
    tpb = _NPARTS_TC // _NPARTS_SC

    tids = token_ids.astype(jnp.int32)
    seg_pad = jnp.pad(segment_table, ((0, 8 - segment_table.shape[0]), (0, 0)))
    sid3 = segment_ids.astype(jnp.int32).reshape(n // s_chunk, 1, s_chunk)
    gamma2d = ln_gamma.reshape(1, d)
    beta2d = ln_beta.reshape(1, d)

    toks = [
        _sc_gather_part(
            token_table,
            lax.slice(tids, (0, g * sc_chunk), (b, (g + 1) * sc_chunk)))
        for g in range(_NPARTS_SC)
    ]
    out = None
    for h in range(_NPARTS_TC):
        out = _tc_add_layernorm_part(
            toks[h // tpb], position_table, seg_pad, sid3, gamma2d, beta2d,
            1e-5, out, h, s_chunk, n, b,
        )
    return out.reshape(b, s, d)


# speedup vs baseline: 1.2319x; 1.2319x over previous
import functools

import jax
import jax.numpy as jnp
from jax import lax
from jax.experimental import pallas as pl
from jax.experimental.pallas import tpu as pltpu
from jax.experimental.pallas import tpu_sc as plsc

_NC = 2
_NS = 16
_NW = _NC * _NS

_NPARTS_SC = 2
_NPARTS_TC = 4
_GATHER_CHUNK = 64
_NBUF = 2
_DEPTH = 2


def _sc_gather_part(table, tids_part):
    b_sz, s_chunk = tids_part.shape
    d = table.shape[1]
    n_p = b_sz * s_chunk
    rpw = n_p // _NW
    wpb = _NW // b_sz
    ch = min(_GATHER_CHUNK, rpw)
    nch = rpw // ch
    assert rpw * _NW == n_p and ch * nch == rpw

    mesh = plsc.VectorSubcoreMesh(
        core_axis_name="c", subcore_axis_name="s",
        num_cores=_NC, num_subcores=_NS,
    )

    @functools.partial(
        pl.kernel,
        mesh=mesh,
        out_type=jax.ShapeDtypeStruct((n_p, d), jnp.float32),
        scratch_types=(
            [pltpu.VMEM((rpw,), jnp.int32)]
            + [pltpu.VMEM((ch, d), jnp.float32) for _ in range(_NBUF)]
            + [pltpu.SemaphoreType.DMA for _ in range(2 * _NBUF)]
        ),
    )
    def k(table_hbm, tids_hbm, out_hbm, idx_v, *bufs):
        rows = bufs[:_NBUF]
        gsem = bufs[_NBUF:2 * _NBUF]
        ssem = bufs[2 * _NBUF:]
        wid = lax.axis_index("s") * _NC + lax.axis_index("c")
        bi = wid // wpb
        si = (wid % wpb) * rpw
        base = wid * rpw
        pltpu.sync_copy(tids_hbm.at[bi, pl.ds(si, rpw)], idx_v)

        def gather(ci):
            b = ci % _NBUF
            return pltpu.async_copy(
                table_hbm.at[idx_v.at[pl.ds(ci * ch, ch)]], rows[b], gsem[b])

        def store(ci):
            b = ci % _NBUF
            return pltpu.async_copy(
                rows[b], out_hbm.at[pl.ds(base + ci * ch, ch)], ssem[b])

        depth = min(_DEPTH, nch)
        gcp = [None] * _NBUF
        scp = [None] * _NBUF
        for ci in range(depth):
            gcp[ci % _NBUF] = gather(ci)
        for ci in range(nch):
            b = ci % _NBUF
            gcp[b].wait()
            scp[b] = store(ci)
            nxt = ci + depth
            if nxt < nch:
                nb = nxt % _NBUF
                if scp[nb] is not None:
                    scp[nb].wait()
                gcp[nb] = gather(nxt)
        for ci in range(max(0, nch - _NBUF), nch):
            scp[ci % _NBUF].wait()

    return k(table, tids_part)


def _tc_add_layernorm_part(tok, pos_table, seg_table, sid3, gamma2d, beta2d,
                           eps, prev, h, s_chunk, total_n, b_sz):
    n_p, d = tok.shape
    s_len = pos_table.shape[0]
    br = s_chunk
    assert n_p % br == 0
    sc_chunk = n_p // b_sz
    tpb = sc_chunk // br
    spb = s_len // br
    nblk = b_sz

    def out_map(i):
        return (i * spb + h, 0)

    def body(tok_ref, pos_ref, seg_ref, sid_ref, g_ref, b_ref, *rest):
        o_ref = rest[-1]
        s0 = seg_ref[0, :]
        sd = seg_ref[1, :] - s0
        sid_col = sid_ref[0, 0, :].astype(jnp.float32).reshape(br, 1)
        x = tok_ref[...] + pos_ref[...] + s0[None, :] + sid_col * sd[None, :]
        mean = jnp.mean(x, axis=-1, keepdims=True)
        xc = x - mean
        var = jnp.mean(xc * xc, axis=-1, keepdims=True)
        inv = lax.rsqrt(var + eps)
        o_ref[...] = xc * inv * g_ref[...] + b_ref[...]

    in_specs = [
        pl.BlockSpec((br, d), lambda i: (i * tpb + h % tpb, 0)),
        pl.BlockSpec((br, d), lambda i: (h, 0)),
        pl.BlockSpec((8, d), lambda i: (0, 0)),
        pl.BlockSpec((1, 1, br), lambda i: (out_map(i)[0], 0, 0)),
        pl.BlockSpec((1, d), lambda i: (0, 0)),
        pl.BlockSpec((1, d), lambda i: (0, 0)),
    ]
    args = [tok, pos_table, seg_table, sid3, gamma2d, beta2d]
    aliases = {}
    if prev is not None:
        in_specs.append(pl.BlockSpec(memory_space=pl.ANY))
        args.append(prev)
        aliases = {6: 0}

    return pl.pallas_call(
        body,
        grid=(nblk,),
        in_specs=in_specs,
        out_specs=pl.BlockSpec((br, d), out_map),
        out_shape=jax.ShapeDtypeStruct((total_n, d), jnp.float32),
        input_output_aliases=aliases,
    )(*args)


def kernel(token_ids, segment_ids, input_ids, token_table, segment_table,
           position_table, ln_gamma, ln_beta):
    b, s = input_ids.shape
    d = token_table.shape[1]
    n = b * s
    sc_chunk = s // _NPARTS_SC
    s_chunk = s // _NPARTS_TC
    tpb = _NPARTS_TC // _NPARTS_SC

    tids = token_ids.astype(jnp.int32)
    seg_pad = jnp.pad(segment_table, ((0, 8 - segment_table.shape[0]), (0, 0)))
    sid3 = segment_ids.astype(jnp.int32).reshape(n // s_chunk, 1, s_chunk)
    gamma2d = ln_gamma.reshape(1, d)
    beta2d = ln_beta.reshape(1, d)

    toks = [
        _sc_gather_part(
            token_table,
            lax.slice(tids, (0, g * sc_chunk), (b, (g + 1) * sc_chunk)))
        for g in range(_NPARTS_SC)
    ]
    out = None
    for h in range(_NPARTS_TC):
        out = _tc_add_layernorm_part(
            toks[h // tpb], position_table, seg_pad, sid3, gamma2d, beta2d,
            1e-5, out, h, s_chunk, n, b,
        )
    return out.reshape(b, s, d)
